# baseline (speedup 1.0000x reference)
import functools

import jax
import jax.numpy as jnp
from jax import lax
from jax.experimental import pallas as pl
from jax.experimental.pallas import tpu as pltpu
from jax.experimental.pallas import tpu_sc as plsc

_N = 10000
_E = 320000
_D = 128
_G = 64
_HID = 256
_OUT = 64

_NT = 16
_NW = 32
_CH = 128
_NCH = 79
_WIN = _NCH * _CH
_NB = 80
_GK = 10
_NGRP = _NB // _GK
_CROWS = _N * _G
_CPT = _CROWS // _NT
_DROWS = 10240
_DPT = _DROWS // _NT
_ZR = 8000


def _edge_histogram(edge_in, batch_in):
    mesh = plsc.VectorSubcoreMesh(core_axis_name="c", subcore_axis_name="s")
    out_type = (
        jax.ShapeDtypeStruct((_CROWS,), jnp.float32),
        jax.ShapeDtypeStruct((_CROWS,), jnp.float32),
        jax.ShapeDtypeStruct((_DROWS,), jnp.float32),
        jax.ShapeDtypeStruct((_DROWS,), jnp.float32),
    )
    scratch = [
        pltpu.VMEM((2, _WIN), jnp.int32),
        pltpu.VMEM((_NB, _CH), jnp.int32),
        pltpu.VMEM((_NB, _CH), jnp.int32),
        pltpu.VMEM((_NB, _CH), jnp.int32),
        pltpu.VMEM((_NB, _CH), jnp.float32),
        pltpu.VMEM((_ZR,), jnp.float32),
        pltpu.VMEM((_ZR,), jnp.float32),
        pltpu.VMEM_SHARED((_N,), jnp.int32),
        pltpu.VMEM_SHARED((_CROWS,), jnp.float32),
        pltpu.VMEM_SHARED((_DROWS,), jnp.float32),
        pltpu.SemaphoreType.DMA,
        pltpu.SemaphoreType.DMA,
        pltpu.SemaphoreType.DMA,
    ]

    @functools.partial(pl.kernel, out_type=out_type, mesh=mesh,
                       scratch_types=scratch)
    def k(edge_h, batch_h, c0_h, c1_h, d0_h, d1_h,
          ed_v, g_v, idxc_v, idxd_v, val_v, bufa_v, bufb_v,
          batch_sh, c_sh, d_sh, semg, sems, semz):
        cid = lax.axis_index("c")
        sid = lax.axis_index("s")
        wid = cid * _NT + sid
        lo = wid * _WIN
        hi = jnp.minimum(lo + _WIN, _E)
        start = pl.multiple_of(jnp.minimum(lo, _E - _WIN), _CH)

        edcp = pltpu.async_copy(edge_h.at[:, pl.ds(start, _WIN)], ed_v,
                                semg)

        @pl.when(sid == 0)
        def _():
            pltpu.sync_copy(batch_h, batch_sh)

        def zfill(i, carry):
            bufa_v[pl.ds(i * 16, 16)] = jnp.zeros((16,), jnp.float32)
            return carry
        lax.fori_loop(0, _ZR // 16, zfill, 0)

        nz = _CPT // _ZR
        zcps = [
            pltpu.async_copy(
                bufa_v, c_sh.at[pl.ds(sid * _CPT + t * _ZR, _ZR)], semz)
            for t in range(nz)
        ]
        zcps.append(pltpu.async_copy(
            bufa_v.at[pl.ds(0, _DPT)], d_sh.at[pl.ds(sid * _DPT, _DPT)],
            semz))
        for cp in zcps:
            cp.wait()
        edcp.wait()
        plsc.subcore_barrier()

        lane = jnp.arange(16, dtype=jnp.int32)

        def g_descr(grp, i, make):
            b = grp * _GK + i
            base = jnp.minimum(b * _CH, _WIN - _CH)
            src = batch_sh.at[ed_v.at[1, pl.ds(base, _CH)]]
            if make:
                pltpu.async_copy(src, g_v.at[b], semg)
            else:
                pltpu.make_async_copy(src, g_v.at[b], semg).wait()

        def s_descr(grp, i, make):
            b = grp * _GK + i
            if make:
                pltpu.async_copy(val_v.at[b], c_sh.at[idxc_v.at[b]],
                                 sems, add=True)
                pltpu.async_copy(val_v.at[b], d_sh.at[idxd_v.at[b]],
                                 sems, add=True)
            else:
                pltpu.make_async_copy(val_v.at[b], c_sh.at[idxc_v.at[b]],
                                      sems).wait()
                pltpu.make_async_copy(val_v.at[b], d_sh.at[idxd_v.at[b]],
                                      sems).wait()

        for i in range(_GK):
            g_descr(0, i, True)

        def body(grp, carry):
            for i in range(_GK):
                g_descr(grp, i, False)

            @pl.when(grp < _NGRP - 1)
            def _():
                for i in range(_GK):
                    g_descr(grp + 1, i, True)

            for i in range(_GK):
                b = grp * _GK + i
                base = jnp.minimum(b * _CH, _WIN - _CH)
                for kk in range(_CH // 16):
                    sl = pl.ds(base + kk * 16, 16)
                    osl = pl.ds(kk * 16, 16)
                    gv = g_v[b, osl]
                    srcv = ed_v[0, sl]
                    dstv = ed_v[1, sl]
                    local = b * _CH + kk * 16 + lane
                    glob = start + local
                    valid = ((local < _WIN) & (glob >= lo) & (glob < hi))
                    idxc_v[b, osl] = srcv * _G + gv
                    idxd_v[b, osl] = dstv
                    val_v[b, osl] = jnp.where(
                        valid, 1.0, 0.0).astype(jnp.float32)

            @pl.when(grp > 0)
            def _():
                for i in range(_GK):
                    s_descr(grp - 1, i, False)

            for i in range(_GK):
                s_descr(grp, i, True)
            return carry
        lax.fori_loop(0, _NGRP, body, 0)
        for i in range(_GK):
            s_descr(_NGRP - 1, i, False)
        plsc.subcore_barrier()

        def writeout(c_h, d_h):
            bufs = [bufa_v, bufb_v]
            rd = []
            for t in range(nz):
                sl = pl.ds(sid * _CPT + t * _ZR, _ZR)
                rd.append(pltpu.async_copy(c_sh.at[sl], bufs[t % 2], semg))
            wr = []
            for t in range(nz):
                sl = pl.ds(sid * _CPT + t * _ZR, _ZR)
                if t >= 2:
                    wr[t - 2].wait()
                rd[t].wait()
                wr.append(pltpu.async_copy(bufs[t % 2], c_h.at[sl], sems))
            dsl = pl.ds(sid * _DPT, _DPT)
            pltpu.sync_copy(d_sh.at[dsl], bufs[nz % 2].at[pl.ds(0, _DPT)])
            wr.append(pltpu.async_copy(
                bufs[nz % 2].at[pl.ds(0, _DPT)], d_h.at[dsl], sems))
            for cp in wr[-3:]:
                cp.wait()

        @pl.when(cid == 0)
        def _():
            writeout(c0_h, d0_h)

        @pl.when(cid == 1)
        def _():
            writeout(c1_h, d1_h)

    return k(edge_in, batch_in)


def _dot(a, b, dn):
    return lax.dot_general(a, b, (dn, ((), ())),
                           preferred_element_type=jnp.float32)


def _dense_body(x_r, m0_r, m1_r, d0_r, d1_r, b_r, wphi_r, bphi_r,
                wg_r, bg_r, w1_r, b1_r, w2_r, b2_r, o_r):
    f32 = jnp.float32

    x = x_r[...]
    M = m0_r[...] + m1_r[...]
    Ce = M[:, :_G]
    Co = M[:, _G:]
    x3 = x.reshape(_N // 2, 2, _D)
    xe = x3[:, 0, :]
    xo = x3[:, 1, :]
    degrow = (d0_r[...] + d1_r[...])[:, :_N]
    batch = b_r[...]
    iota = lax.broadcasted_iota(jnp.int32, (_G, _N), 0)
    onehot = (batch == iota).astype(f32)
    ohdeg = onehot * degrow

    pS = _dot(Ce, xe, ((0,), (0,))) + _dot(Co, xo, ((0,), (0,)))
    px = _dot(onehot, x, ((1,), (0,)))
    T = _dot(ohdeg, x, ((1,), (0,)))
    counts = jnp.sum(onehot, axis=1, keepdims=True)
    Eg = jnp.sum(ohdeg, axis=1, keepdims=True)

    A = wphi_r[0:_D, :]
    B = wphi_r[_D:, :]
    Wg1 = wg_r[0:_D, :]
    Wg2 = wg_r[_D:, :]

    pa = (_dot(T, A, ((1,), (0,))) + _dot(pS, B, ((1,), (0,)))
          + Eg * bphi_r[...])
    num = (_dot(px, Wg1, ((1,), (0,))) + _dot(pa, Wg2, ((1,), (0,)))
           + counts * bg_r[...])
    pooled = num / jnp.maximum(counts, 1.0)
    hid = jnp.maximum(_dot(pooled, w1_r[...], ((1,), (0,))) + b1_r[...], 0.0)
    o_r[...] = _dot(hid, w2_r[...], ((1,), (0,))) + b2_r[...]


def kernel(x, edge_index, batch, Wphi, bphi, Wgamma, bgamma, W1, b1, W2, b2):
    c0, c1, d0, d1 = _edge_histogram(edge_index, batch)

    m0 = c0.reshape(_N // 2, 128)
    m1 = c1.reshape(_N // 2, 128)
    d0 = d0.reshape(1, _DROWS)
    d1 = d1.reshape(1, _DROWS)

    out = pl.pallas_call(
        _dense_body,
        out_shape=jax.ShapeDtypeStruct((_G, _OUT), jnp.float32),
    )(x, m0, m1, d0, d1, batch.reshape(1, _N),
      Wphi, bphi.reshape(1, _D), Wgamma, bgamma.reshape(1, _D),
      W1, b1.reshape(1, _HID), W2, b2.reshape(1, _OUT))
    return out


# speedup vs baseline: 1.0135x; 1.0135x over previous
import functools

import jax
import jax.numpy as jnp
from jax import lax
from jax.experimental import pallas as pl
from jax.experimental.pallas import tpu as pltpu
from jax.experimental.pallas import tpu_sc as plsc

_N = 10000
_E = 320000
_D = 128
_G = 64
_HID = 256
_OUT = 64

_NT = 16
_NW = 32
_CH = 128
_NCH = 79
_WIN = _NCH * _CH
_NB = 80
_GK = 8
_NGRP = _NB // _GK
_CROWS = _N * _G
_CPT = _CROWS // _NT
_DROWS = 10240
_DPT = _DROWS // _NT
_ZR = 8000


def _edge_histogram(edge_in, batch_in):
    mesh = plsc.VectorSubcoreMesh(core_axis_name="c", subcore_axis_name="s")
    out_type = (
        jax.ShapeDtypeStruct((_CROWS,), jnp.float32),
        jax.ShapeDtypeStruct((_CROWS,), jnp.float32),
        jax.ShapeDtypeStruct((_DROWS,), jnp.float32),
        jax.ShapeDtypeStruct((_DROWS,), jnp.float32),
    )
    scratch = [
        pltpu.VMEM((2, _WIN), jnp.int32),
        pltpu.VMEM((_NB, _CH), jnp.int32),
        pltpu.VMEM((_NB, _CH), jnp.int32),
        pltpu.VMEM((_NB, _CH), jnp.int32),
        pltpu.VMEM((_NB, _CH), jnp.float32),
        pltpu.VMEM((_ZR,), jnp.float32),
        pltpu.VMEM((_ZR,), jnp.float32),
        pltpu.VMEM_SHARED((_N,), jnp.int32),
        pltpu.VMEM_SHARED((_CROWS,), jnp.float32),
        pltpu.VMEM_SHARED((_DROWS,), jnp.float32),
        pltpu.SemaphoreType.DMA,
        pltpu.SemaphoreType.DMA,
        pltpu.SemaphoreType.DMA,
    ]

    @functools.partial(pl.kernel, out_type=out_type, mesh=mesh,
                       scratch_types=scratch)
    def k(edge_h, batch_h, c0_h, c1_h, d0_h, d1_h,
          ed_v, g_v, idxc_v, idxd_v, val_v, bufa_v, bufb_v,
          batch_sh, c_sh, d_sh, semg, sems, semz):
        cid = lax.axis_index("c")
        sid = lax.axis_index("s")
        wid = cid * _NT + sid
        lo = wid * _WIN
        hi = jnp.minimum(lo + _WIN, _E)
        start = pl.multiple_of(jnp.minimum(lo, _E - _WIN), _CH)

        edcp = pltpu.async_copy(edge_h.at[:, pl.ds(start, _WIN)], ed_v,
                                semg)

        @pl.when(sid == 0)
        def _():
            pltpu.sync_copy(batch_h, batch_sh)

        def zfill(i, carry):
            bufa_v[pl.ds(i * 16, 16)] = jnp.zeros((16,), jnp.float32)
            return carry
        lax.fori_loop(0, _ZR // 16, zfill, 0)

        nz = _CPT // _ZR
        zcps = [
            pltpu.async_copy(
                bufa_v, c_sh.at[pl.ds(sid * _CPT + t * _ZR, _ZR)], semz)
            for t in range(nz)
        ]
        zcps.append(pltpu.async_copy(
            bufa_v.at[pl.ds(0, _DPT)], d_sh.at[pl.ds(sid * _DPT, _DPT)],
            semz))
        for cp in zcps:
            cp.wait()
        edcp.wait()
        plsc.subcore_barrier()

        lane = jnp.arange(16, dtype=jnp.int32)

        def g_descr(grp, i, make):
            b = grp * _GK + i
            base = jnp.minimum(b * _CH, _WIN - _CH)
            src = batch_sh.at[ed_v.at[1, pl.ds(base, _CH)]]
            if make:
                pltpu.async_copy(src, g_v.at[b], semg)
            else:
                pltpu.make_async_copy(src, g_v.at[b], semg).wait()

        def s_descr(grp, i, make):
            b = grp * _GK + i
            if make:
                pltpu.async_copy(val_v.at[b], c_sh.at[idxc_v.at[b]],
                                 sems, add=True)
                pltpu.async_copy(val_v.at[b], d_sh.at[idxd_v.at[b]],
                                 sems, add=True)
            else:
                pltpu.make_async_copy(val_v.at[b], c_sh.at[idxc_v.at[b]],
                                      sems).wait()
                pltpu.make_async_copy(val_v.at[b], d_sh.at[idxd_v.at[b]],
                                      sems).wait()

        for i in range(_GK):
            g_descr(0, i, True)

        def body(grp, carry):
            for i in range(_GK):
                g_descr(grp, i, False)

            @pl.when(grp < _NGRP - 1)
            def _():
                for i in range(_GK):
                    g_descr(grp + 1, i, True)

            for i in range(_GK):
                b = grp * _GK + i
                base = jnp.minimum(b * _CH, _WIN - _CH)
                for kk in range(_CH // 16):
                    sl = pl.ds(base + kk * 16, 16)
                    osl = pl.ds(kk * 16, 16)
                    gv = g_v[b, osl]
                    srcv = ed_v[0, sl]
                    dstv = ed_v[1, sl]
                    local = b * _CH + kk * 16 + lane
                    glob = start + local
                    valid = ((local < _WIN) & (glob >= lo) & (glob < hi))
                    idxc_v[b, osl] = srcv * _G + gv
                    idxd_v[b, osl] = dstv
                    val_v[b, osl] = jnp.where(
                        valid, 1.0, 0.0).astype(jnp.float32)

            @pl.when(grp > 0)
            def _():
                for i in range(_GK):
                    s_descr(grp - 1, i, False)

            for i in range(_GK):
                s_descr(grp, i, True)
            return carry
        lax.fori_loop(0, _NGRP, body, 0)
        for i in range(_GK):
            s_descr(_NGRP - 1, i, False)
        plsc.subcore_barrier()

        def writeout(c_h, d_h):
            bufs = [bufa_v, bufb_v]
            rd = []
            for t in range(nz):
                sl = pl.ds(sid * _CPT + t * _ZR, _ZR)
                rd.append(pltpu.async_copy(c_sh.at[sl], bufs[t % 2], semg))
            wr = []
            for t in range(nz):
                sl = pl.ds(sid * _CPT + t * _ZR, _ZR)
                if t >= 2:
                    wr[t - 2].wait()
                rd[t].wait()
                wr.append(pltpu.async_copy(bufs[t % 2], c_h.at[sl], sems))
            dsl = pl.ds(sid * _DPT, _DPT)
            pltpu.sync_copy(d_sh.at[dsl], bufs[nz % 2].at[pl.ds(0, _DPT)])
            wr.append(pltpu.async_copy(
                bufs[nz % 2].at[pl.ds(0, _DPT)], d_h.at[dsl], sems))
            for cp in wr[-3:]:
                cp.wait()

        @pl.when(cid == 0)
        def _():
            writeout(c0_h, d0_h)

        @pl.when(cid == 1)
        def _():
            writeout(c1_h, d1_h)

    return k(edge_in, batch_in)


def _dot(a, b, dn):
    return lax.dot_general(a, b, (dn, ((), ())),
                           preferred_element_type=jnp.float32)


def _dense_body(x_r, m0_r, m1_r, d0_r, d1_r, b_r, wphi_r, bphi_r,
                wg_r, bg_r, w1_r, b1_r, w2_r, b2_r, o_r):
    f32 = jnp.float32

    x = x_r[...]
    M = m0_r[...] + m1_r[...]
    Ce = M[:, :_G]
    Co = M[:, _G:]
    x3 = x.reshape(_N // 2, 2, _D)
    xe = x3[:, 0, :]
    xo = x3[:, 1, :]
    degrow = (d0_r[...] + d1_r[...])[:, :_N]
    batch = b_r[...]
    iota = lax.broadcasted_iota(jnp.int32, (_G, _N), 0)
    onehot = (batch == iota).astype(f32)
    ohdeg = onehot * degrow

    pS = _dot(Ce, xe, ((0,), (0,))) + _dot(Co, xo, ((0,), (0,)))
    px = _dot(onehot, x, ((1,), (0,)))
    T = _dot(ohdeg, x, ((1,), (0,)))
    counts = jnp.sum(onehot, axis=1, keepdims=True)
    Eg = jnp.sum(ohdeg, axis=1, keepdims=True)

    A = wphi_r[0:_D, :]
    B = wphi_r[_D:, :]
    Wg1 = wg_r[0:_D, :]
    Wg2 = wg_r[_D:, :]

    pa = (_dot(T, A, ((1,), (0,))) + _dot(pS, B, ((1,), (0,)))
          + Eg * bphi_r[...])
    num = (_dot(px, Wg1, ((1,), (0,))) + _dot(pa, Wg2, ((1,), (0,)))
           + counts * bg_r[...])
    pooled = num / jnp.maximum(counts, 1.0)
    hid = jnp.maximum(_dot(pooled, w1_r[...], ((1,), (0,))) + b1_r[...], 0.0)
    o_r[...] = _dot(hid, w2_r[...], ((1,), (0,))) + b2_r[...]


def kernel(x, edge_index, batch, Wphi, bphi, Wgamma, bgamma, W1, b1, W2, b2):
    c0, c1, d0, d1 = _edge_histogram(edge_index, batch)

    m0 = c0.reshape(_N // 2, 128)
    m1 = c1.reshape(_N // 2, 128)
    d0 = d0.reshape(1, _DROWS)
    d1 = d1.reshape(1, _DROWS)

    out = pl.pallas_call(
        _dense_body,
        out_shape=jax.ShapeDtypeStruct((_G, _OUT), jnp.float32),
    )(x, m0, m1, d0, d1, batch.reshape(1, _N),
      Wphi, bphi.reshape(1, _D), Wgamma, bgamma.reshape(1, _D),
      W1, b1.reshape(1, _HID), W2, b2.reshape(1, _OUT))
    return out
